# baseline (speedup 1.0000x reference)
import functools

import jax
import jax.numpy as jnp
from jax import lax
from jax.experimental import pallas as pl
from jax.experimental.pallas import tpu as pltpu
from jax.experimental.pallas import tpu_sc as plsc

NCLUST = 10
DIN = 1024
DHID = 512
DATT = 256
NTOK = 50000
BLK = 256
NBLK = 208
NPAD = NBLK * BLK
NTOKP = 50176
CHUNK = NTOKP // 32


def _tc_kernel(blk_cid_ref, blk_valid_ref,
               x_ref, w1_ref, b1_ref, w2_ref, b2_ref, counts_ref,
               wfc_ref, bfc_ref, wa_ref, ba_ref, wb_ref, bb_ref,
               wc_ref, bc_ref, wrho_ref, brho_ref, wcls_ref, bcls_ref,
               logits_ref, prob_ref, yhat_ref, acc_ref):
    i = pl.program_id(0)
    cid = blk_cid_ref[i]
    nvalid = blk_valid_ref[i]

    @pl.when(i == 0)
    def _init():
        acc_ref[...] = jnp.zeros_like(acc_ref)

    x = x_ref[...]
    h = jnp.dot(x, w1_ref[cid], preferred_element_type=jnp.float32)
    h = jnp.maximum(h + b1_ref[pl.ds(cid, 1), :], 0.0)
    h = jnp.dot(h, w2_ref[cid], preferred_element_type=jnp.float32)
    h = jnp.maximum(h + b2_ref[pl.ds(cid, 1), :], 0.0)
    rows = lax.broadcasted_iota(jnp.int32, (BLK, 1), 0)
    h = jnp.where(rows < nvalid, h, 0.0)
    part = jnp.sum(h, axis=0, keepdims=True)
    acc_ref[pl.ds(cid, 1), :] += part

    @pl.when(i == NBLK - 1)
    def _head():
        lane10 = lax.broadcasted_iota(jnp.int32, (16, 1), 0) < NCLUST
        h_cluster = acc_ref[...] / counts_ref[...]
        h_path = jnp.dot(h_cluster, wfc_ref[...], preferred_element_type=jnp.float32)
        h_path = jnp.maximum(h_path + bfc_ref[...], 0.0)
        a = jnp.tanh(jnp.dot(h_path, wa_ref[...], preferred_element_type=jnp.float32) + ba_ref[...])
        b = jax.nn.sigmoid(jnp.dot(h_path, wb_ref[...], preferred_element_type=jnp.float32) + bb_ref[...])
        g = a * b
        scores = jnp.sum(g * wc_ref[...], axis=1, keepdims=True) + bc_ref[0, 0]
        scores = jnp.where(lane10, scores, -jnp.inf)
        m = jnp.max(scores, axis=0, keepdims=True)
        e = jnp.exp(scores - m)
        attn = e / jnp.sum(e, axis=0, keepdims=True)
        hp = jnp.sum(attn * h_path, axis=0, keepdims=True)
        hr = jnp.dot(hp, wrho_ref[...], preferred_element_type=jnp.float32)
        hr = jnp.maximum(hr + brho_ref[...], 0.0)
        logits = jnp.dot(hr, wcls_ref[...], preferred_element_type=jnp.float32) + bcls_ref[...]
        logits_ref[...] = logits
        lm = jnp.max(logits, axis=1, keepdims=True)
        le = jnp.exp(logits - lm)
        prob_ref[...] = le / jnp.sum(le, axis=1, keepdims=True)
        lane4 = lax.broadcasted_iota(jnp.int32, (1, 4), 1)
        yhat_ref[...] = jnp.min(
            jnp.where(logits >= lm, lane4, 9999), axis=1, keepdims=True)


def _run_tc(x_sorted, blk_cid, blk_valid, counts16, phi_W1, phi_b1, phi_W2,
            phi_b2, W_fc, b_fc, W_a, b_a, W_b, b_b, W_c, b_c, W_rho, b_rho,
            W_cls, b_cls):
    full = lambda *shape: pl.BlockSpec(shape, lambda i, *_: (0,) * len(shape))
    grid_spec = pltpu.PrefetchScalarGridSpec(
        num_scalar_prefetch=2,
        grid=(NBLK,),
        in_specs=[
            pl.BlockSpec((BLK, DIN), lambda i, *_: (i, 0)),
            full(NCLUST, DIN, DHID),
            full(NCLUST, DHID),
            full(NCLUST, DHID, DHID),
            full(NCLUST, DHID),
            full(16, 1),
            full(DHID, DHID), full(1, DHID),
            full(DHID, DATT), full(1, DATT),
            full(DHID, DATT), full(1, DATT),
            full(1, DATT), full(1, 1),
            full(DHID, DATT), full(1, DATT),
            full(DATT, 4), full(1, 4),
        ],
        out_specs=[full(1, 4), full(1, 4), full(1, 1)],
        scratch_shapes=[pltpu.VMEM((16, DHID), jnp.float32)],
    )
    return pl.pallas_call(
        _tc_kernel,
        grid_spec=grid_spec,
        out_shape=[
            jax.ShapeDtypeStruct((1, 4), jnp.float32),
            jax.ShapeDtypeStruct((1, 4), jnp.float32),
            jax.ShapeDtypeStruct((1, 1), jnp.int32),
        ],
    )(blk_cid, blk_valid, x_sorted, phi_W1, phi_b1, phi_W2, phi_b2, counts16,
      W_fc, b_fc.reshape(1, DHID), W_a, b_a.reshape(1, DATT),
      W_b, b_b.reshape(1, DATT), W_c.reshape(1, DATT), b_c.reshape(1, 1),
      W_rho, b_rho.reshape(1, DATT), W_cls, b_cls.reshape(1, 4))


ROWS_W = NPAD // 32
GCH = 104
NGCH = ROWS_W // GCH
DW = DIN // 2


def _sc_gather_body(data_hbm, perm_hbm, out_hbm, idx_v, rows_v,
                    gsem0, gsem1, wsem0, wsem1):
    wid = lax.axis_index("s") * 2 + lax.axis_index("c")
    base = wid * ROWS_W
    pltpu.sync_copy(perm_hbm.at[pl.ds(base, ROWS_W)], idx_v)

    def clamp(k, _):
        o = pl.multiple_of(k * 16, 16)
        v = idx_v[pl.ds(o, 16)]
        idx_v[pl.ds(o, 16)] = jnp.clip(v, 0, NTOK - 1)
        return 0
    lax.fori_loop(0, ROWS_W // 16, clamp, 0)

    gsems = (gsem0, gsem1)
    wsems = (wsem0, wsem1)

    def start_gather(j):
        return pltpu.async_copy(
            data_hbm.at[idx_v.at[pl.ds(j * GCH, GCH)]],
            rows_v.at[j % 2], gsems[j % 2])

    gathers = {0: start_gather(0)}
    writes = {}
    for j in range(NGCH):
        b = j % 2
        gathers.pop(j).wait()
        writes[j] = pltpu.async_copy(
            rows_v.at[b], out_hbm.at[pl.ds(base + j * GCH, GCH)], wsems[b])
        if j + 1 < NGCH:
            if j - 1 in writes:
                writes.pop(j - 1).wait()
            gathers[j + 1] = start_gather(j + 1)
    writes.pop(NGCH - 1).wait()


def _run_sc_gather(data_i32, perm):
    mesh = plsc.VectorSubcoreMesh(core_axis_name="c", subcore_axis_name="s")
    return pl.kernel(
        _sc_gather_body,
        out_type=jax.ShapeDtypeStruct((NPAD, DW), jnp.int32),
        mesh=mesh,
        scratch_types=[
            pltpu.VMEM((ROWS_W,), jnp.int32),
            pltpu.VMEM((2, GCH, DW), jnp.int32),
            pltpu.SemaphoreType.DMA,
            pltpu.SemaphoreType.DMA,
            pltpu.SemaphoreType.DMA,
            pltpu.SemaphoreType.DMA,
        ],
    )(data_i32, perm)


def _route_host(cluster_id):
    cid = cluster_id.astype(jnp.int32)
    counts = jnp.zeros((NCLUST,), jnp.int32).at[cid].add(1)
    padded = ((counts + BLK - 1) // BLK) * BLK
    offsets = jnp.concatenate([jnp.zeros((1,), jnp.int32), jnp.cumsum(padded)])
    order = jnp.argsort(cid, stable=True)
    sorted_start = jnp.concatenate([jnp.zeros((1,), jnp.int32),
                                    jnp.cumsum(counts)])
    cid_sorted = cid[order]
    dest = (offsets[cid_sorted] + jnp.arange(NTOK, dtype=jnp.int32)
            - sorted_start[cid_sorted])
    perm = jnp.zeros((NPAD,), jnp.int32).at[dest].set(order.astype(jnp.int32))
    jb = jnp.arange(NBLK, dtype=jnp.int32) * BLK
    blk_cid = jnp.sum((jb[:, None] >= offsets[None, 1:]).astype(jnp.int32),
                      axis=1)
    blk_cid = jnp.minimum(blk_cid, NCLUST - 1)
    blk_valid = jnp.clip(counts[blk_cid] - (jb - offsets[blk_cid]), 0, BLK)
    counts16 = jnp.where(jnp.arange(16) < NCLUST,
                         jnp.concatenate([counts, jnp.ones((6,), jnp.int32)]),
                         1).astype(jnp.float32).reshape(16, 1)
    return perm, blk_cid, blk_valid, counts16


def kernel(data, cluster_id, phi_W1, phi_b1, phi_W2, phi_b2, W_fc, b_fc,
           W_a, b_a, W_b, b_b, W_c, b_c, W_rho, b_rho, W_cls, b_cls):
    perm, blk_cid, blk_valid, counts16 = _route_host(cluster_id)
    data_i32 = lax.bitcast_convert_type(
        data.astype(jnp.bfloat16).reshape(NTOK, DW, 2), jnp.int32)
    sorted_i32 = _run_sc_gather(data_i32, perm)
    x_sorted = lax.bitcast_convert_type(
        sorted_i32, jnp.bfloat16).reshape(NPAD, DIN)
    logits, prob, yhat = _run_tc(
        x_sorted, blk_cid, blk_valid, counts16,
        phi_W1.astype(jnp.bfloat16), phi_b1,
        phi_W2.astype(jnp.bfloat16), phi_b2,
        W_fc, b_fc, W_a, b_a, W_b, b_b, W_c, b_c, W_rho, b_rho,
        W_cls, b_cls)
    return (logits, prob, yhat)


# speedup vs baseline: 3.0246x; 3.0246x over previous
import functools

import jax
import jax.numpy as jnp
from jax import lax
from jax.experimental import pallas as pl
from jax.experimental.pallas import tpu as pltpu
from jax.experimental.pallas import tpu_sc as plsc

NCLUST = 10
DIN = 1024
DHID = 512
DATT = 256
NTOK = 50000
BLK = 256
NBLK = 208
NPAD = NBLK * BLK
NTOKP = 50176
CHUNK = NTOKP // 32


def _tc_kernel(blk_cid_ref, blk_valid_ref,
               x_ref, w1_ref, b1_ref, w2_ref, b2_ref, counts_ref,
               wfc_ref, bfc_ref, wa_ref, ba_ref, wb_ref, bb_ref,
               wc_ref, bc_ref, wrho_ref, brho_ref, wcls_ref, bcls_ref,
               logits_ref, prob_ref, yhat_ref, acc_ref):
    i = pl.program_id(0)
    cid = blk_cid_ref[i]
    nvalid = blk_valid_ref[i]

    @pl.when(i == 0)
    def _init():
        acc_ref[...] = jnp.zeros_like(acc_ref)

    x = x_ref[...].astype(jnp.bfloat16)
    h = jnp.dot(x, w1_ref[cid], preferred_element_type=jnp.float32)
    h = jnp.maximum(h + b1_ref[pl.ds(cid, 1), :], 0.0)
    h = jnp.dot(h.astype(jnp.bfloat16), w2_ref[cid],
                preferred_element_type=jnp.float32)
    h = jnp.maximum(h + b2_ref[pl.ds(cid, 1), :], 0.0)
    rows = lax.broadcasted_iota(jnp.int32, (BLK, 1), 0)
    h = jnp.where(rows < nvalid, h, 0.0)
    part = jnp.sum(h, axis=0, keepdims=True)
    acc_ref[pl.ds(cid, 1), :] += part

    @pl.when(i == NBLK - 1)
    def _head():
        lane10 = lax.broadcasted_iota(jnp.int32, (16, 1), 0) < NCLUST
        h_cluster = acc_ref[...] / counts_ref[...]
        h_path = jnp.dot(h_cluster, wfc_ref[...], preferred_element_type=jnp.float32)
        h_path = jnp.maximum(h_path + bfc_ref[...], 0.0)
        a = jnp.tanh(jnp.dot(h_path, wa_ref[...], preferred_element_type=jnp.float32) + ba_ref[...])
        b = jax.nn.sigmoid(jnp.dot(h_path, wb_ref[...], preferred_element_type=jnp.float32) + bb_ref[...])
        g = a * b
        scores = jnp.sum(g * wc_ref[...], axis=1, keepdims=True) + bc_ref[0, 0]
        scores = jnp.where(lane10, scores, -jnp.inf)
        m = jnp.max(scores, axis=0, keepdims=True)
        e = jnp.exp(scores - m)
        attn = e / jnp.sum(e, axis=0, keepdims=True)
        hp = jnp.sum(attn * h_path, axis=0, keepdims=True)
        hr = jnp.dot(hp, wrho_ref[...], preferred_element_type=jnp.float32)
        hr = jnp.maximum(hr + brho_ref[...], 0.0)
        logits = jnp.dot(hr, wcls_ref[...], preferred_element_type=jnp.float32) + bcls_ref[...]
        logits_ref[...] = logits
        lm = jnp.max(logits, axis=1, keepdims=True)
        le = jnp.exp(logits - lm)
        prob_ref[...] = le / jnp.sum(le, axis=1, keepdims=True)
        lane4 = lax.broadcasted_iota(jnp.int32, (1, 4), 1)
        yhat_ref[...] = jnp.min(
            jnp.where(logits >= lm, lane4, 9999), axis=1, keepdims=True)


def _run_tc(x_sorted, blk_cid, blk_valid, counts16, phi_W1, phi_b1, phi_W2,
            phi_b2, W_fc, b_fc, W_a, b_a, W_b, b_b, W_c, b_c, W_rho, b_rho,
            W_cls, b_cls):
    full = lambda *shape: pl.BlockSpec(shape, lambda i, *_: (0,) * len(shape))
    grid_spec = pltpu.PrefetchScalarGridSpec(
        num_scalar_prefetch=2,
        grid=(NBLK,),
        in_specs=[
            pl.BlockSpec((BLK, DIN), lambda i, *_: (i, 0)),
            full(NCLUST, DIN, DHID),
            full(NCLUST, DHID),
            full(NCLUST, DHID, DHID),
            full(NCLUST, DHID),
            full(16, 1),
            full(DHID, DHID), full(1, DHID),
            full(DHID, DATT), full(1, DATT),
            full(DHID, DATT), full(1, DATT),
            full(1, DATT), full(1, 1),
            full(DHID, DATT), full(1, DATT),
            full(DATT, 4), full(1, 4),
        ],
        out_specs=[full(1, 4), full(1, 4), full(1, 1)],
        scratch_shapes=[pltpu.VMEM((16, DHID), jnp.float32)],
    )
    return pl.pallas_call(
        _tc_kernel,
        grid_spec=grid_spec,
        out_shape=[
            jax.ShapeDtypeStruct((1, 4), jnp.float32),
            jax.ShapeDtypeStruct((1, 4), jnp.float32),
            jax.ShapeDtypeStruct((1, 1), jnp.int32),
        ],
    )(blk_cid, blk_valid, x_sorted, phi_W1, phi_b1, phi_W2, phi_b2, counts16,
      W_fc, b_fc.reshape(1, DHID), W_a, b_a.reshape(1, DATT),
      W_b, b_b.reshape(1, DATT), W_c.reshape(1, DATT), b_c.reshape(1, 1),
      W_rho, b_rho.reshape(1, DATT), W_cls, b_cls.reshape(1, 4))


ROWS_W = NPAD // 32
GCH = 32
NGCH = ROWS_W // GCH


def _sc_gather_body(data_hbm, perm_hbm, out_hbm, idx_v, rows_v,
                    gsem0, gsem1, wsem0, wsem1):
    wid = lax.axis_index("s") * 2 + lax.axis_index("c")
    base = wid * ROWS_W
    pltpu.sync_copy(perm_hbm.at[pl.ds(base, ROWS_W)], idx_v)

    def clamp(k, _):
        o = pl.multiple_of(k * 16, 16)
        v = idx_v[pl.ds(o, 16)]
        idx_v[pl.ds(o, 16)] = jnp.clip(v, 0, NTOK - 1)
        return 0
    lax.fori_loop(0, ROWS_W // 16, clamp, 0)

    gsems = (gsem0, gsem1)
    wsems = (wsem0, wsem1)

    def start_gather(j):
        return pltpu.async_copy(
            data_hbm.at[idx_v.at[pl.ds(j * GCH, GCH)]],
            rows_v.at[j % 2], gsems[j % 2])

    gathers = {0: start_gather(0)}
    writes = {}
    for j in range(NGCH):
        b = j % 2
        gathers.pop(j).wait()
        writes[j] = pltpu.async_copy(
            rows_v.at[b], out_hbm.at[pl.ds(base + j * GCH, GCH)], wsems[b])
        if j + 1 < NGCH:
            if j - 1 in writes:
                writes.pop(j - 1).wait()
            gathers[j + 1] = start_gather(j + 1)
    writes.pop(NGCH - 1).wait()


def _run_sc_gather(data, perm):
    mesh = plsc.VectorSubcoreMesh(core_axis_name="c", subcore_axis_name="s")
    return pl.kernel(
        _sc_gather_body,
        out_type=jax.ShapeDtypeStruct((NPAD, DIN), jnp.float32),
        mesh=mesh,
        scratch_types=[
            pltpu.VMEM((ROWS_W,), jnp.int32),
            pltpu.VMEM((2, GCH, DIN), jnp.float32),
            pltpu.SemaphoreType.DMA,
            pltpu.SemaphoreType.DMA,
            pltpu.SemaphoreType.DMA,
            pltpu.SemaphoreType.DMA,
        ],
    )(data, perm)


def _route_host(cluster_id):
    cid = cluster_id.astype(jnp.int32)
    counts = jnp.zeros((NCLUST,), jnp.int32).at[cid].add(1)
    padded = ((counts + BLK - 1) // BLK) * BLK
    offsets = jnp.concatenate([jnp.zeros((1,), jnp.int32), jnp.cumsum(padded)])
    order = jnp.argsort(cid, stable=True)
    sorted_start = jnp.concatenate([jnp.zeros((1,), jnp.int32),
                                    jnp.cumsum(counts)])
    cid_sorted = cid[order]
    dest = (offsets[cid_sorted] + jnp.arange(NTOK, dtype=jnp.int32)
            - sorted_start[cid_sorted])
    perm = jnp.zeros((NPAD,), jnp.int32).at[dest].set(order.astype(jnp.int32))
    jb = jnp.arange(NBLK, dtype=jnp.int32) * BLK
    blk_cid = jnp.sum((jb[:, None] >= offsets[None, 1:]).astype(jnp.int32),
                      axis=1)
    blk_cid = jnp.minimum(blk_cid, NCLUST - 1)
    blk_valid = jnp.clip(counts[blk_cid] - (jb - offsets[blk_cid]), 0, BLK)
    counts16 = jnp.where(jnp.arange(16) < NCLUST,
                         jnp.concatenate([counts, jnp.ones((6,), jnp.int32)]),
                         1).astype(jnp.float32).reshape(16, 1)
    return perm, blk_cid, blk_valid, counts16


def kernel(data, cluster_id, phi_W1, phi_b1, phi_W2, phi_b2, W_fc, b_fc,
           W_a, b_a, W_b, b_b, W_c, b_c, W_rho, b_rho, W_cls, b_cls):
    perm, blk_cid, blk_valid, counts16 = _route_host(cluster_id)
    x_sorted = _run_sc_gather(data, perm)
    logits, prob, yhat = _run_tc(
        x_sorted, blk_cid, blk_valid, counts16,
        phi_W1.astype(jnp.bfloat16), phi_b1,
        phi_W2.astype(jnp.bfloat16), phi_b2,
        W_fc, b_fc, W_a, b_a, W_b, b_b, W_c, b_c, W_rho, b_rho,
        W_cls, b_cls)
    return (logits, prob, yhat)


# speedup vs baseline: 4.0563x; 1.3411x over previous
import functools

import numpy as np

import jax
import jax.numpy as jnp
from jax import lax
from jax.experimental import pallas as pl
from jax.experimental.pallas import tpu as pltpu
from jax.experimental.pallas import tpu_sc as plsc

NCLUST = 10
DIN = 1024
DHID = 512
DATT = 256
NTOK = 50000
BLK = 256
NBLK = 208
NPAD = NBLK * BLK
NTOKP = 50176
CHUNK = NTOKP // 32


def _tc_kernel(blk_cid_ref, blk_valid_ref,
               x_ref, w1_ref, b1_ref, w2_ref, b2_ref, counts_ref,
               wfc_ref, bfc_ref, wa_ref, ba_ref, wb_ref, bb_ref,
               wc_ref, bc_ref, wrho_ref, brho_ref, wcls_ref, bcls_ref,
               logits_ref, prob_ref, yhat_ref, acc_ref):
    i = pl.program_id(0)
    cid = blk_cid_ref[i]
    nvalid = blk_valid_ref[i]

    @pl.when(i == 0)
    def _init():
        acc_ref[...] = jnp.zeros_like(acc_ref)

    x = x_ref[...].astype(jnp.bfloat16)
    h = jnp.dot(x, w1_ref[cid], preferred_element_type=jnp.float32)
    h = jnp.maximum(h + b1_ref[pl.ds(cid, 1), :], 0.0)
    h = jnp.dot(h.astype(jnp.bfloat16), w2_ref[cid],
                preferred_element_type=jnp.float32)
    h = jnp.maximum(h + b2_ref[pl.ds(cid, 1), :], 0.0)
    rows = lax.broadcasted_iota(jnp.int32, (BLK, 1), 0)
    h = jnp.where(rows < nvalid, h, 0.0)
    part = jnp.sum(h, axis=0, keepdims=True)
    acc_ref[pl.ds(cid, 1), :] += part

    @pl.when(i == NBLK - 1)
    def _head():
        lane10 = lax.broadcasted_iota(jnp.int32, (16, 1), 0) < NCLUST
        h_cluster = acc_ref[...] / counts_ref[...]
        h_path = jnp.dot(h_cluster, wfc_ref[...], preferred_element_type=jnp.float32)
        h_path = jnp.maximum(h_path + bfc_ref[...], 0.0)
        a = jnp.tanh(jnp.dot(h_path, wa_ref[...], preferred_element_type=jnp.float32) + ba_ref[...])
        b = jax.nn.sigmoid(jnp.dot(h_path, wb_ref[...], preferred_element_type=jnp.float32) + bb_ref[...])
        g = a * b
        scores = jnp.sum(g * wc_ref[...], axis=1, keepdims=True) + bc_ref[0, 0]
        scores = jnp.where(lane10, scores, -jnp.inf)
        m = jnp.max(scores, axis=0, keepdims=True)
        e = jnp.exp(scores - m)
        attn = e / jnp.sum(e, axis=0, keepdims=True)
        hp = jnp.sum(attn * h_path, axis=0, keepdims=True)
        hr = jnp.dot(hp, wrho_ref[...], preferred_element_type=jnp.float32)
        hr = jnp.maximum(hr + brho_ref[...], 0.0)
        logits = jnp.dot(hr, wcls_ref[...], preferred_element_type=jnp.float32) + bcls_ref[...]
        logits_ref[...] = logits
        lm = jnp.max(logits, axis=1, keepdims=True)
        le = jnp.exp(logits - lm)
        prob_ref[...] = le / jnp.sum(le, axis=1, keepdims=True)
        lane4 = lax.broadcasted_iota(jnp.int32, (1, 4), 1)
        yhat_ref[...] = jnp.min(
            jnp.where(logits >= lm, lane4, 9999), axis=1, keepdims=True)


def _run_tc(x_sorted, blk_cid, blk_valid, counts16, phi_W1, phi_b1, phi_W2,
            phi_b2, W_fc, b_fc, W_a, b_a, W_b, b_b, W_c, b_c, W_rho, b_rho,
            W_cls, b_cls):
    full = lambda *shape: pl.BlockSpec(shape, lambda i, *_: (0,) * len(shape))
    grid_spec = pltpu.PrefetchScalarGridSpec(
        num_scalar_prefetch=2,
        grid=(NBLK,),
        in_specs=[
            pl.BlockSpec((BLK, DIN), lambda i, *_: (i, 0)),
            full(NCLUST, DIN, DHID),
            full(NCLUST, DHID),
            full(NCLUST, DHID, DHID),
            full(NCLUST, DHID),
            full(16, 1),
            full(DHID, DHID), full(1, DHID),
            full(DHID, DATT), full(1, DATT),
            full(DHID, DATT), full(1, DATT),
            full(1, DATT), full(1, 1),
            full(DHID, DATT), full(1, DATT),
            full(DATT, 4), full(1, 4),
        ],
        out_specs=[full(1, 4), full(1, 4), full(1, 1)],
        scratch_shapes=[pltpu.VMEM((16, DHID), jnp.float32)],
    )
    return pl.pallas_call(
        _tc_kernel,
        grid_spec=grid_spec,
        out_shape=[
            jax.ShapeDtypeStruct((1, 4), jnp.float32),
            jax.ShapeDtypeStruct((1, 4), jnp.float32),
            jax.ShapeDtypeStruct((1, 1), jnp.int32),
        ],
    )(blk_cid, blk_valid, x_sorted, phi_W1, phi_b1, phi_W2, phi_b2, counts16,
      W_fc, b_fc.reshape(1, DHID), W_a, b_a.reshape(1, DATT),
      W_b, b_b.reshape(1, DATT), W_c.reshape(1, DATT), b_c.reshape(1, 1),
      W_rho, b_rho.reshape(1, DATT), W_cls, b_cls.reshape(1, 4))


ROWS_W = NPAD // 32
GCH = 32
NGCH = ROWS_W // GCH


def _sc_gather_body(data_hbm, perm_hbm, out_hbm, idx_v, rows_v,
                    gsem0, gsem1, wsem0, wsem1):
    wid = lax.axis_index("s") * 2 + lax.axis_index("c")
    base = wid * ROWS_W
    pltpu.sync_copy(perm_hbm.at[pl.ds(base, ROWS_W)], idx_v)

    def clamp(k, _):
        o = pl.multiple_of(k * 16, 16)
        v = idx_v[pl.ds(o, 16)]
        idx_v[pl.ds(o, 16)] = jnp.clip(v, 0, NTOK - 1)
        return 0
    lax.fori_loop(0, ROWS_W // 16, clamp, 0)

    gsems = (gsem0, gsem1)
    wsems = (wsem0, wsem1)

    def start_gather(j):
        return pltpu.async_copy(
            data_hbm.at[idx_v.at[pl.ds(j * GCH, GCH)]],
            rows_v.at[j % 2], gsems[j % 2])

    gathers = {0: start_gather(0)}
    writes = {}
    for j in range(NGCH):
        b = j % 2
        gathers.pop(j).wait()
        writes[j] = pltpu.async_copy(
            rows_v.at[b], out_hbm.at[pl.ds(base + j * GCH, GCH)], wsems[b])
        if j + 1 < NGCH:
            if j - 1 in writes:
                writes.pop(j - 1).wait()
            gathers[j + 1] = start_gather(j + 1)
    writes.pop(NGCH - 1).wait()


def _run_sc_gather(data, perm):
    mesh = plsc.VectorSubcoreMesh(core_axis_name="c", subcore_axis_name="s")
    return pl.kernel(
        _sc_gather_body,
        out_type=jax.ShapeDtypeStruct((NPAD, DIN), jnp.float32),
        mesh=mesh,
        scratch_types=[
            pltpu.VMEM((ROWS_W,), jnp.int32),
            pltpu.VMEM((2, GCH, DIN), jnp.float32),
            pltpu.SemaphoreType.DMA,
            pltpu.SemaphoreType.DMA,
            pltpu.SemaphoreType.DMA,
            pltpu.SemaphoreType.DMA,
        ],
    )(data, perm)


SUBCH = 112
NSUB = CHUNK // SUBCH


def _lane():
    return lax.iota(jnp.int32, 16)


def _place16(v, bases, lane, tmp_ref):
    del tmp_ref
    pos = jnp.zeros((16,), jnp.int32)
    zero = lane * 0
    for i in range(16):
        cc = v.at[zero + i].get(mode="promise_in_bounds")
        b = bases.at[cc].get(mode="promise_in_bounds")
        pos = pos + jnp.where(lane == i, b, 0)
        bases = bases + jnp.maximum(1 - jnp.abs(lane - cc), 0)
    return pos, bases


def _sc_route_body(cid_hbm, perm_hbm, blkcid_hbm, blkvalid_hbm, counts_hbm,
                   cid_v, pos_v, vals_v, hist_v, hist_all_v,
                   blkcid_v, blkvalid_v, countsf_v, shared_hist):
    c = lax.axis_index("c")
    s = lax.axis_index("s")
    lane = lax.iota(jnp.int32, 16)
    w_own = 2 * s + c

    pltpu.sync_copy(
        cid_hbm.at[pl.ds(pl.multiple_of(s * 2 * CHUNK, 8), 2 * CHUNK)], cid_v)

    for half in range(2):
        def hbody(k, cnt, half=half):
            off = pl.multiple_of(half * CHUNK + k * 16, 16)
            _, cnt = _place16(cid_v[pl.ds(off, 16)], cnt, lane, hist_v)
            return cnt
        cnt = lax.fori_loop(0, CHUNK // 16, hbody, jnp.zeros((16,), jnp.int32))
        hist_v[...] = cnt
        row = pl.multiple_of((2 * s + half) * 16, 16)
        pltpu.sync_copy(hist_v, shared_hist.at[pl.ds(row, 16)])
    plsc.subcore_barrier()
    pltpu.sync_copy(shared_hist, hist_all_v)

    totals = jnp.zeros((16,), jnp.int32)
    prefix = jnp.zeros((16,), jnp.int32)
    for r in range(32):
        h = hist_all_v[pl.ds(r * 16, 16)]
        totals = totals + h
        prefix = prefix + h * (r < w_own).astype(jnp.int32)

    real = lane < NCLUST
    padded = jnp.where(real, ((totals + BLK - 1) >> 8) << 8, 0)
    x = padded
    for d in (1, 2, 4, 8):
        shifted = x.at[jnp.maximum(lane - d, 0)].get(mode="promise_in_bounds")
        x = x + jnp.where(lane >= d, shifted, 0)
    off_ex = x - padded
    bases = jnp.where(real, off_ex + prefix, NPAD + prefix)

    tok0 = w_own * CHUNK

    def place(j, bases):
        for k in range(SUBCH // 16):
            off = pl.multiple_of(c * CHUNK + j * SUBCH + k * 16, 16)
            v = cid_v[pl.ds(off, 16)]
            pos, bases = _place16(v, bases, lane, hist_v)
            pos_v[pl.ds(k * 16, 16)] = pos
            vals_v[pl.ds(k * 16, 16)] = tok0 + j * SUBCH + k * 16 + lane
        pltpu.sync_copy(vals_v, perm_hbm.at[pos_v])
        return bases
    lax.fori_loop(0, NSUB, place, bases)

    @pl.when(jnp.logical_and(c == 0, s == 0))
    def _meta():
        zero = lane * 0
        ends = off_ex + padded
        countsf_v[...] = jnp.where(real, totals, 1).astype(jnp.float32)
        for t in range(NBLK // 16):
            jb = (lane + 16 * t) * BLK
            bcid = jnp.zeros((16,), jnp.int32)
            for cc in range(NCLUST):
                end_cc = ends.at[zero + cc].get(mode="promise_in_bounds")
                bcid = bcid + jnp.clip(jb - end_cc + 1, 0, 1)
            bcid = jnp.minimum(bcid, NCLUST - 1)
            off_g = off_ex.at[bcid].get(mode="promise_in_bounds")
            tot_g = totals.at[bcid].get(mode="promise_in_bounds")
            valid = jnp.clip(tot_g - (jb - off_g), 0, BLK)
            blkcid_v[pl.ds(16 * t, 16)] = bcid
            blkvalid_v[pl.ds(16 * t, 16)] = valid
        pltpu.sync_copy(countsf_v, counts_hbm)
        pltpu.sync_copy(blkcid_v, blkcid_hbm)
        pltpu.sync_copy(blkvalid_v, blkvalid_hbm)


def _run_sc_route(cid_pad):
    mesh = plsc.VectorSubcoreMesh(core_axis_name="c", subcore_axis_name="s")
    perm, blkcid, blkvalid, counts = pl.kernel(
        _sc_route_body,
        out_type=(
            jax.ShapeDtypeStruct((NPAD + 256,), jnp.int32),
            jax.ShapeDtypeStruct((NBLK,), jnp.int32),
            jax.ShapeDtypeStruct((NBLK,), jnp.int32),
            jax.ShapeDtypeStruct((16,), jnp.float32),
        ),
        mesh=mesh,
        scratch_types=[
            pltpu.VMEM((2 * CHUNK,), jnp.int32),
            pltpu.VMEM((SUBCH,), jnp.int32),
            pltpu.VMEM((SUBCH,), jnp.int32),
            pltpu.VMEM((16,), jnp.int32),
            pltpu.VMEM((512,), jnp.int32),
            pltpu.VMEM((NBLK,), jnp.int32),
            pltpu.VMEM((NBLK,), jnp.int32),
            pltpu.VMEM((16,), jnp.float32),
            pltpu.VMEM_SHARED((512,), jnp.int32),
        ],
    )(cid_pad)
    return perm, blkcid, blkvalid, counts


def kernel(data, cluster_id, phi_W1, phi_b1, phi_W2, phi_b2, W_fc, b_fc,
           W_a, b_a, W_b, b_b, W_c, b_c, W_rho, b_rho, W_cls, b_cls):
    cid_pad = jnp.concatenate([
        cluster_id.astype(jnp.int32),
        jnp.full((NTOKP - NTOK,), NCLUST, jnp.int32)])
    perm, blk_cid, blk_valid, counts = _run_sc_route(cid_pad)
    counts16 = counts.reshape(16, 1)
    x_sorted = _run_sc_gather(data, perm)
    logits, prob, yhat = _run_tc(
        x_sorted, blk_cid, blk_valid, counts16,
        phi_W1.astype(jnp.bfloat16), phi_b1,
        phi_W2.astype(jnp.bfloat16), phi_b2,
        W_fc, b_fc, W_a, b_a, W_b, b_b, W_c, b_c, W_rho, b_rho,
        W_cls, b_cls)
    return (logits, prob, yhat)


# speedup vs baseline: 4.5908x; 1.1318x over previous
import functools

import numpy as np

import jax
import jax.numpy as jnp
from jax import lax
from jax.experimental import pallas as pl
from jax.experimental.pallas import tpu as pltpu
from jax.experimental.pallas import tpu_sc as plsc

NCLUST = 10
DIN = 1024
DHID = 512
DATT = 256
NTOK = 50000
BLK = 256
NBLK = 208
NPAD = NBLK * BLK
NTOKP = 50176
CHUNK = NTOKP // 32


NSTRIPE = 4
SBLK = NBLK // NSTRIPE


def _tc_kernel(blk_cid_ref, blk_valid_ref,
               x_ref, w1_ref, b1_ref, w2_ref, b2_ref,
               acc_out_ref, acc_ref):
    i = pl.program_id(0)
    cid = blk_cid_ref[i]
    nvalid = blk_valid_ref[i]

    @pl.when(i == 0)
    def _init():
        acc_ref[...] = jnp.zeros_like(acc_ref)

    x = x_ref[...].astype(jnp.bfloat16)
    h = jnp.dot(x, w1_ref[cid], preferred_element_type=jnp.float32)
    h = jnp.maximum(h + b1_ref[pl.ds(cid, 1), :], 0.0)
    h = jnp.dot(h.astype(jnp.bfloat16), w2_ref[cid],
                preferred_element_type=jnp.float32)
    h = jnp.maximum(h + b2_ref[pl.ds(cid, 1), :], 0.0)
    rows = lax.broadcasted_iota(jnp.int32, (BLK, 1), 0)
    h = jnp.where(rows < nvalid, h, 0.0)
    part = jnp.sum(h, axis=0, keepdims=True)
    acc_ref[pl.ds(cid, 1), :] += part

    @pl.when(i == SBLK - 1)
    def _flush():
        acc_out_ref[...] = acc_ref[...]


def _head_kernel(accs_ref, counts_ref, wfc_ref, bfc_ref, wa_ref, ba_ref,
                 wb_ref, bb_ref, wc_ref, bc_ref, wrho_ref, brho_ref,
                 wcls_ref, bcls_ref, logits_ref, prob_ref, yhat_ref):
    lane10 = lax.broadcasted_iota(jnp.int32, (16, 1), 0) < NCLUST
    acc = (accs_ref[0] + accs_ref[1]) + (accs_ref[2] + accs_ref[3])
    h_cluster = acc / counts_ref[...]
    h_path = jnp.dot(h_cluster, wfc_ref[...], preferred_element_type=jnp.float32)
    h_path = jnp.maximum(h_path + bfc_ref[...], 0.0)
    a = jnp.tanh(jnp.dot(h_path, wa_ref[...], preferred_element_type=jnp.float32) + ba_ref[...])
    b = jax.nn.sigmoid(jnp.dot(h_path, wb_ref[...], preferred_element_type=jnp.float32) + bb_ref[...])
    g = a * b
    scores = jnp.sum(g * wc_ref[...], axis=1, keepdims=True) + bc_ref[0, 0]
    scores = jnp.where(lane10, scores, -jnp.inf)
    m = jnp.max(scores, axis=0, keepdims=True)
    e = jnp.exp(scores - m)
    attn = e / jnp.sum(e, axis=0, keepdims=True)
    hp = jnp.sum(attn * h_path, axis=0, keepdims=True)
    hr = jnp.dot(hp, wrho_ref[...], preferred_element_type=jnp.float32)
    hr = jnp.maximum(hr + brho_ref[...], 0.0)
    logits = jnp.dot(hr, wcls_ref[...], preferred_element_type=jnp.float32) + bcls_ref[...]
    logits_ref[...] = logits
    lm = jnp.max(logits, axis=1, keepdims=True)
    le = jnp.exp(logits - lm)
    prob_ref[...] = le / jnp.sum(le, axis=1, keepdims=True)
    lane4 = lax.broadcasted_iota(jnp.int32, (1, 4), 1)
    yhat_ref[...] = jnp.min(
        jnp.where(logits >= lm, lane4, 9999), axis=1, keepdims=True)


def _run_tc_stripe(x_stripe, blk_cid_s, blk_valid_s, phi_W1, phi_b1,
                   phi_W2, phi_b2):
    full = lambda *shape: pl.BlockSpec(shape, lambda i, *_: (0,) * len(shape))
    grid_spec = pltpu.PrefetchScalarGridSpec(
        num_scalar_prefetch=2,
        grid=(SBLK,),
        in_specs=[
            pl.BlockSpec((BLK, DIN), lambda i, *_: (i, 0)),
            full(NCLUST, DIN, DHID),
            full(NCLUST, DHID),
            full(NCLUST, DHID, DHID),
            full(NCLUST, DHID),
        ],
        out_specs=[full(16, DHID)],
        scratch_shapes=[pltpu.VMEM((16, DHID), jnp.float32)],
    )
    return pl.pallas_call(
        _tc_kernel,
        grid_spec=grid_spec,
        out_shape=[jax.ShapeDtypeStruct((16, DHID), jnp.float32)],
    )(blk_cid_s, blk_valid_s, x_stripe, phi_W1, phi_b1, phi_W2, phi_b2)[0]


def _run_head(accs, counts16, W_fc, b_fc, W_a, b_a, W_b, b_b, W_c, b_c,
              W_rho, b_rho, W_cls, b_cls):
    full = lambda *shape: pl.BlockSpec(shape, lambda *_: (0,) * len(shape))
    return pl.pallas_call(
        _head_kernel,
        in_specs=[
            full(NSTRIPE, 16, DHID), full(16, 1),
            full(DHID, DHID), full(1, DHID),
            full(DHID, DATT), full(1, DATT),
            full(DHID, DATT), full(1, DATT),
            full(1, DATT), full(1, 1),
            full(DHID, DATT), full(1, DATT),
            full(DATT, 4), full(1, 4),
        ],
        out_specs=[full(1, 4), full(1, 4), full(1, 1)],
        out_shape=[
            jax.ShapeDtypeStruct((1, 4), jnp.float32),
            jax.ShapeDtypeStruct((1, 4), jnp.float32),
            jax.ShapeDtypeStruct((1, 1), jnp.int32),
        ],
    )(accs, counts16,
      W_fc, b_fc.reshape(1, DHID), W_a, b_a.reshape(1, DATT),
      W_b, b_b.reshape(1, DATT), W_c.reshape(1, DATT), b_c.reshape(1, 1),
      W_rho, b_rho.reshape(1, DATT), W_cls, b_cls.reshape(1, 4))


NROWS_S = NPAD // 4
ROWS_W = NROWS_S // 32
GCH = 32
NGCH = ROWS_W // GCH


def _sc_gather_body(q, data_hbm, perm_hbm, out_hbm, idx_v, rows_v,
                    gsem0, gsem1, wsem0, wsem1):
    wid = lax.axis_index("s") * 2 + lax.axis_index("c")
    base = wid * ROWS_W
    pltpu.sync_copy(perm_hbm.at[pl.ds(q * NROWS_S + base, ROWS_W)], idx_v)

    def clamp(k, _):
        o = pl.multiple_of(k * 16, 16)
        v = idx_v[pl.ds(o, 16)]
        idx_v[pl.ds(o, 16)] = jnp.clip(v, 0, NTOK - 1)
        return 0
    lax.fori_loop(0, ROWS_W // 16, clamp, 0)

    gsems = (gsem0, gsem1)
    wsems = (wsem0, wsem1)

    def start_gather(j):
        return pltpu.async_copy(
            data_hbm.at[idx_v.at[pl.ds(j * GCH, GCH)]],
            rows_v.at[j % 2], gsems[j % 2])

    gathers = {0: start_gather(0)}
    writes = {}
    for j in range(NGCH):
        b = j % 2
        gathers.pop(j).wait()
        writes[j] = pltpu.async_copy(
            rows_v.at[b], out_hbm.at[pl.ds(base + j * GCH, GCH)], wsems[b])
        if j + 1 < NGCH:
            if j - 1 in writes:
                writes.pop(j - 1).wait()
            gathers[j + 1] = start_gather(j + 1)
    writes.pop(NGCH - 1).wait()


def _run_sc_gather(data, perm, q):
    mesh = plsc.VectorSubcoreMesh(core_axis_name="c", subcore_axis_name="s")
    return pl.kernel(
        functools.partial(_sc_gather_body, q),
        out_type=jax.ShapeDtypeStruct((NROWS_S, DIN), jnp.float32),
        mesh=mesh,
        scratch_types=[
            pltpu.VMEM((ROWS_W,), jnp.int32),
            pltpu.VMEM((2, GCH, DIN), jnp.float32),
            pltpu.SemaphoreType.DMA,
            pltpu.SemaphoreType.DMA,
            pltpu.SemaphoreType.DMA,
            pltpu.SemaphoreType.DMA,
        ],
    )(data, perm)


SUBCH = 112
NSUB = CHUNK // SUBCH


def _lane():
    return lax.iota(jnp.int32, 16)


def _place16(v, bases, lane, tmp_ref):
    del tmp_ref
    pos = jnp.zeros((16,), jnp.int32)
    zero = lane * 0
    for i in range(16):
        cc = v.at[zero + i].get(mode="promise_in_bounds")
        b = bases.at[cc].get(mode="promise_in_bounds")
        pos = pos + jnp.where(lane == i, b, 0)
        bases = bases + jnp.maximum(1 - jnp.abs(lane - cc), 0)
    return pos, bases


def _sc_route_body(cid_hbm, perm_hbm, blkcid_hbm, blkvalid_hbm, counts_hbm,
                   cid_v, pos_v, vals_v, hist_v, hist_all_v,
                   blkcid_v, blkvalid_v, countsf_v, shared_hist):
    c = lax.axis_index("c")
    s = lax.axis_index("s")
    lane = lax.iota(jnp.int32, 16)
    w_own = 2 * s + c

    pltpu.sync_copy(
        cid_hbm.at[pl.ds(pl.multiple_of(s * 2 * CHUNK, 8), 2 * CHUNK)], cid_v)

    for half in range(2):
        def hbody(k, cnt, half=half):
            off = pl.multiple_of(half * CHUNK + k * 16, 16)
            _, cnt = _place16(cid_v[pl.ds(off, 16)], cnt, lane, hist_v)
            return cnt
        cnt = lax.fori_loop(0, CHUNK // 16, hbody, jnp.zeros((16,), jnp.int32))
        hist_v[...] = cnt
        row = pl.multiple_of((2 * s + half) * 16, 16)
        pltpu.sync_copy(hist_v, shared_hist.at[pl.ds(row, 16)])
    plsc.subcore_barrier()
    pltpu.sync_copy(shared_hist, hist_all_v)

    totals = jnp.zeros((16,), jnp.int32)
    prefix = jnp.zeros((16,), jnp.int32)
    for r in range(32):
        h = hist_all_v[pl.ds(r * 16, 16)]
        totals = totals + h
        prefix = prefix + h * (r < w_own).astype(jnp.int32)

    real = lane < NCLUST
    padded = jnp.where(real, ((totals + BLK - 1) >> 8) << 8, 0)
    x = padded
    for d in (1, 2, 4, 8):
        shifted = x.at[jnp.maximum(lane - d, 0)].get(mode="promise_in_bounds")
        x = x + jnp.where(lane >= d, shifted, 0)
    off_ex = x - padded
    bases = jnp.where(real, off_ex + prefix, NPAD + prefix)

    tok0 = w_own * CHUNK

    def place(j, bases):
        for k in range(SUBCH // 16):
            off = pl.multiple_of(c * CHUNK + j * SUBCH + k * 16, 16)
            v = cid_v[pl.ds(off, 16)]
            pos, bases = _place16(v, bases, lane, hist_v)
            pos_v[pl.ds(k * 16, 16)] = pos
            vals_v[pl.ds(k * 16, 16)] = tok0 + j * SUBCH + k * 16 + lane
        pltpu.sync_copy(vals_v, perm_hbm.at[pos_v])
        return bases
    lax.fori_loop(0, NSUB, place, bases)

    @pl.when(jnp.logical_and(c == 0, s == 0))
    def _meta():
        zero = lane * 0
        ends = off_ex + padded
        countsf_v[...] = jnp.where(real, totals, 1).astype(jnp.float32)
        for t in range(NBLK // 16):
            jb = (lane + 16 * t) * BLK
            bcid = jnp.zeros((16,), jnp.int32)
            for cc in range(NCLUST):
                end_cc = ends.at[zero + cc].get(mode="promise_in_bounds")
                bcid = bcid + jnp.clip(jb - end_cc + 1, 0, 1)
            bcid = jnp.minimum(bcid, NCLUST - 1)
            off_g = off_ex.at[bcid].get(mode="promise_in_bounds")
            tot_g = totals.at[bcid].get(mode="promise_in_bounds")
            valid = jnp.clip(tot_g - (jb - off_g), 0, BLK)
            blkcid_v[pl.ds(16 * t, 16)] = bcid
            blkvalid_v[pl.ds(16 * t, 16)] = valid
        pltpu.sync_copy(countsf_v, counts_hbm)
        pltpu.sync_copy(blkcid_v, blkcid_hbm)
        pltpu.sync_copy(blkvalid_v, blkvalid_hbm)


def _run_sc_route(cid_pad):
    mesh = plsc.VectorSubcoreMesh(core_axis_name="c", subcore_axis_name="s")
    perm, blkcid, blkvalid, counts = pl.kernel(
        _sc_route_body,
        out_type=(
            jax.ShapeDtypeStruct((NPAD + 256,), jnp.int32),
            jax.ShapeDtypeStruct((NBLK,), jnp.int32),
            jax.ShapeDtypeStruct((NBLK,), jnp.int32),
            jax.ShapeDtypeStruct((16,), jnp.float32),
        ),
        mesh=mesh,
        scratch_types=[
            pltpu.VMEM((2 * CHUNK,), jnp.int32),
            pltpu.VMEM((SUBCH,), jnp.int32),
            pltpu.VMEM((SUBCH,), jnp.int32),
            pltpu.VMEM((16,), jnp.int32),
            pltpu.VMEM((512,), jnp.int32),
            pltpu.VMEM((NBLK,), jnp.int32),
            pltpu.VMEM((NBLK,), jnp.int32),
            pltpu.VMEM((16,), jnp.float32),
            pltpu.VMEM_SHARED((512,), jnp.int32),
        ],
    )(cid_pad)
    return perm, blkcid, blkvalid, counts


def kernel(data, cluster_id, phi_W1, phi_b1, phi_W2, phi_b2, W_fc, b_fc,
           W_a, b_a, W_b, b_b, W_c, b_c, W_rho, b_rho, W_cls, b_cls):
    cid_pad = jnp.concatenate([
        cluster_id.astype(jnp.int32),
        jnp.full((NTOKP - NTOK,), NCLUST, jnp.int32)])
    perm, blk_cid, blk_valid, counts = _run_sc_route(cid_pad)
    counts16 = counts.reshape(16, 1)
    w1 = phi_W1.astype(jnp.bfloat16)
    w2 = phi_W2.astype(jnp.bfloat16)
    accs = []
    for q in range(NSTRIPE):
        x_stripe = _run_sc_gather(data, perm, q)
        accs.append(_run_tc_stripe(
            x_stripe, lax.slice(blk_cid, (q * SBLK,), ((q + 1) * SBLK,)),
            lax.slice(blk_valid, (q * SBLK,), ((q + 1) * SBLK,)),
            w1, phi_b1, w2, phi_b2))
    logits, prob, yhat = _run_head(
        jnp.stack(accs), counts16, W_fc, b_fc, W_a, b_a, W_b, b_b,
        W_c, b_c, W_rho, b_rho, W_cls, b_cls)
    return (logits, prob, yhat)
